# Initial kernel scaffold; baseline (speedup 1.0000x reference)
import jax
import jax.numpy as jnp
from jax.experimental import pallas as pl


def kernel(score1, score2, w_vis_mask2, homo12):
    raise NotImplementedError("write your pallas kernel here")


# speedup vs baseline: 30.5286x; 30.5286x over previous
import functools

import numpy as np
import jax
import jax.numpy as jnp
from jax import lax
from jax.experimental import pallas as pl
from jax.experimental.pallas import tpu as pltpu
from jax.experimental.pallas import tpu_sc as plsc

B, H, W = 8, 240, 320
HW = H * W
NMS_K = 5
TOP_K = 512
G_K = 5
G_SIGMA = 0.5
LAM = 1.0

NC, NS = 2, 16
NW = NC * NS
TILES_PER_IMG = NW // B
ROWS_PER_TILE = H // TILES_PER_IMG
PIX_PER_TILE = ROWS_PER_TILE * W
VECS_PER_ROW = W // 16
VECS_PER_TILE = PIX_PER_TILE // 16


def _warp_body(score2_hbm, homo_hbm, out_hbm, img_v, homo_v, out_v):
    cid = lax.axis_index("c")
    sid = lax.axis_index("s")
    wid = sid * NC + cid
    b = wid // TILES_PER_IMG
    quarter = wid % TILES_PER_IMG

    pltpu.sync_copy(score2_hbm.at[b], img_v)
    pltpu.sync_copy(homo_hbm.at[b], homo_v)

    lane = lax.iota(jnp.int32, 16)

    hv = homo_v[...]
    h00, h01, h02 = hv[0], hv[1], hv[2]
    h10, h11, h12 = hv[3], hv[4], hv[5]
    h20, h21, h22 = hv[6], hv[7], hv[8]

    r0 = quarter * ROWS_PER_TILE
    lanef = lane.astype(jnp.float32)

    def body(i, carry):
        y = r0 + i // VECS_PER_ROW
        xb = (i % VECS_PER_ROW) * 16
        yf = y.astype(jnp.float32)
        xf = xb.astype(jnp.float32) + lanef
        z = h20 * xf + (h21 * yf + h22)
        z = jnp.where(jnp.abs(z) < 1e-8, jnp.float32(1e-8), z)
        x2 = (h00 * xf + (h01 * yf + h02)) / z
        y2 = (h10 * xf + (h11 * yf + h12)) / z
        valid = (x2 >= 0.0) & (x2 < W - 1.0) & (y2 >= 0.0) & (y2 < H - 1.0)
        xs = jnp.clip(x2, 0.0, jnp.float32(W - 1))
        ys = jnp.clip(y2, 0.0, jnp.float32(H - 1))
        x0 = xs.astype(jnp.int32)
        y0 = ys.astype(jnp.int32)
        wx = xs - x0.astype(jnp.float32)
        wy = ys - y0.astype(jnp.float32)
        x1 = jnp.minimum(x0 + 1, W - 1)
        y1 = jnp.minimum(y0 + 1, H - 1)
        ra = y0 * W
        rb = y1 * W
        va = plsc.load_gather(img_v, [ra + x0])
        vb = plsc.load_gather(img_v, [ra + x1])
        vc = plsc.load_gather(img_v, [rb + x0])
        vd = plsc.load_gather(img_v, [rb + x1])
        owx = 1.0 - wx
        owy = 1.0 - wy
        val = (va * owx + vb * wx) * owy + (vc * owx + vd * wx) * wy
        val = jnp.where(valid, val, 0.0)
        out_v[pl.ds(i * 16, 16)] = val
        return carry

    lax.fori_loop(0, VECS_PER_TILE, body, 0)
    pltpu.sync_copy(out_v, out_hbm.at[pl.ds(wid * PIX_PER_TILE, PIX_PER_TILE)])


@jax.jit
def _sc_warp(score2_flat, homo_pad):
    mesh = plsc.VectorSubcoreMesh(core_axis_name="c", subcore_axis_name="s",
                                  num_cores=NC, num_subcores=NS)
    return pl.kernel(
        _warp_body,
        out_type=jax.ShapeDtypeStruct((B * HW,), jnp.float32),
        mesh=mesh,
        scratch_types=[
            pltpu.VMEM((HW,), jnp.float32),
            pltpu.VMEM((16,), jnp.float32),
            pltpu.VMEM((PIX_PER_TILE,), jnp.float32),
        ],
        compiler_params=pltpu.CompilerParams(needs_layout_passes=False),
    )(score2_flat, homo_pad)


def _shift(a, axis, s, fill):
    if s == 0:
        return a
    f = jnp.full_like(a, fill)
    n = a.shape[axis]
    if axis == 0:
        if s > 0:
            return jnp.concatenate([a[s:, :], f[:s, :]], axis=0)
        return jnp.concatenate([f[s:, :], a[:n + s, :]], axis=0)
    else:
        if s > 0:
            return jnp.concatenate([a[:, s:], f[:, :s]], axis=1)
        return jnp.concatenate([f[:, s:], a[:, :n + s]], axis=1)


def _win5_max(a, axis):
    neg = jnp.float32(-jnp.inf)
    out = a
    for s in (-2, -1, 1, 2):
        out = jnp.maximum(out, _shift(a, axis, s, neg))
    return out


def _gauss_taps():
    ax = np.arange(G_K, dtype=np.float32) - (G_K - 1) / 2.0
    g = np.exp(-(ax ** 2) / (2.0 * np.float32(G_SIGMA) ** 2)).astype(np.float32)
    g = g / g.sum()
    return [float(v) for v in g]


def _blur1(a, axis, taps):
    out = a * taps[2]
    for k, s in ((0, -2), (1, -1), (3, 1), (4, 2)):
        out = out + _shift(a, axis, s, 0.0) * taps[k]
    return out


def _tc_body(s1_ref, w2_ref, m_ref, out_ref):
    taps = _gauss_taps()
    num = jnp.float32(0.0)
    den = jnp.float32(0.0)
    for b in range(B):
        w2 = w2_ref[b]
        pooled = _win5_max(_win5_max(w2, 1), 0)
        peak = (w2 == pooled) & (w2 > 0.0)
        masked = jnp.where(peak, w2, jnp.float32(0.0))
        bits = lax.bitcast_convert_type(masked, jnp.int32)

        def bisect(i, lohi):
            lo, hi = lohi
            mid = (lo + hi) // 2
            cnt = jnp.sum((bits >= mid).astype(jnp.int32))
            big = cnt >= TOP_K
            return (jnp.where(big, mid, lo), jnp.where(big, hi, mid))

        lo0 = jnp.int32(0)
        hi0 = jnp.int32(0x3F800001)
        lo, hi = lax.fori_loop(0, 26, bisect, (lo0, hi0))
        gt = ((bits >= lo) & (masked > 0.0)).astype(jnp.float32)
        g = _blur1(_blur1(gt, 1, taps), 0, taps)
        d = s1_ref[b] - g
        m = m_ref[b]
        num = num + jnp.sum(d * d * m)
        den = den + jnp.sum(m)
    out_ref[0, 0] = num * LAM / den


@jax.jit
def _tc_rest(s1, w2, mf):
    return pl.pallas_call(
        _tc_body,
        out_shape=jax.ShapeDtypeStruct((1, 1), jnp.float32),
        out_specs=pl.BlockSpec(memory_space=pltpu.SMEM),
    )(s1, w2, mf)


def kernel(score1, score2, w_vis_mask2, homo12):
    s2f = score2.reshape(B, HW)
    homo_pad = jnp.concatenate(
        [homo12.reshape(B, 9), jnp.zeros((B, 7), jnp.float32)], axis=1)
    w_flat = _sc_warp(s2f, homo_pad)
    w2 = w_flat.reshape(B, H, W)
    s1 = score1.reshape(B, H, W)
    mf = w_vis_mask2.astype(jnp.float32).reshape(B, H, W)
    loss = _tc_rest(s1, w2, mf)
    return loss[0, 0]


# speedup vs baseline: 41.8497x; 1.3708x over previous
import functools

import numpy as np
import jax
import jax.numpy as jnp
from jax import lax
from jax.experimental import pallas as pl
from jax.experimental.pallas import tpu as pltpu
from jax.experimental.pallas import tpu_sc as plsc

B, H, W = 8, 240, 320
HW = H * W
NMS_K = 5
TOP_K = 512
G_K = 5
G_SIGMA = 0.5
LAM = 1.0

NC, NS = 2, 16
NW = NC * NS
TILES_PER_IMG = NW // B
ROWS_PER_TILE = H // TILES_PER_IMG
PIX_PER_TILE = ROWS_PER_TILE * W
VECS_PER_ROW = W // 16
VECS_PER_TILE = PIX_PER_TILE // 16


def _warp_body(score2_hbm, homo_hbm, out_hbm, img_v, homo_v, out_v):
    cid = lax.axis_index("c")
    sid = lax.axis_index("s")
    wid = sid * NC + cid
    b = wid // TILES_PER_IMG
    quarter = wid % TILES_PER_IMG

    pltpu.sync_copy(score2_hbm.at[b], img_v)
    pltpu.sync_copy(homo_hbm.at[b], homo_v)

    lane = lax.iota(jnp.int32, 16)

    hv = homo_v[...]
    h00, h01, h02 = hv[0], hv[1], hv[2]
    h10, h11, h12 = hv[3], hv[4], hv[5]
    h20, h21, h22 = hv[6], hv[7], hv[8]

    r0 = quarter * ROWS_PER_TILE
    lanef = lane.astype(jnp.float32)

    def body(i, carry):
        y = r0 + i // VECS_PER_ROW
        xb = (i % VECS_PER_ROW) * 16
        yf = y.astype(jnp.float32)
        xf = xb.astype(jnp.float32) + lanef
        z = h20 * xf + (h21 * yf + h22)
        z = jnp.where(jnp.abs(z) < 1e-8, jnp.float32(1e-8), z)
        x2 = (h00 * xf + (h01 * yf + h02)) / z
        y2 = (h10 * xf + (h11 * yf + h12)) / z
        valid = (x2 >= 0.0) & (x2 < W - 1.0) & (y2 >= 0.0) & (y2 < H - 1.0)
        xs = jnp.clip(x2, 0.0, jnp.float32(W - 1))
        ys = jnp.clip(y2, 0.0, jnp.float32(H - 1))
        x0 = xs.astype(jnp.int32)
        y0 = ys.astype(jnp.int32)
        wx = xs - x0.astype(jnp.float32)
        wy = ys - y0.astype(jnp.float32)
        x1 = jnp.minimum(x0 + 1, W - 1)
        y1 = jnp.minimum(y0 + 1, H - 1)
        ra = y0 * W
        rb = y1 * W
        va = plsc.load_gather(img_v, [ra + x0])
        vb = plsc.load_gather(img_v, [ra + x1])
        vc = plsc.load_gather(img_v, [rb + x0])
        vd = plsc.load_gather(img_v, [rb + x1])
        owx = 1.0 - wx
        owy = 1.0 - wy
        val = (va * owx + vb * wx) * owy + (vc * owx + vd * wx) * wy
        val = jnp.where(valid, val, 0.0)
        out_v[pl.ds(i * 16, 16)] = val
        return carry

    lax.fori_loop(0, VECS_PER_TILE, body, 0)
    pltpu.sync_copy(out_v, out_hbm.at[pl.ds(wid * PIX_PER_TILE, PIX_PER_TILE)])


@jax.jit
def _sc_warp(score2_flat, homo_pad):
    mesh = plsc.VectorSubcoreMesh(core_axis_name="c", subcore_axis_name="s",
                                  num_cores=NC, num_subcores=NS)
    return pl.kernel(
        _warp_body,
        out_type=jax.ShapeDtypeStruct((B * HW,), jnp.float32),
        mesh=mesh,
        scratch_types=[
            pltpu.VMEM((HW,), jnp.float32),
            pltpu.VMEM((16,), jnp.float32),
            pltpu.VMEM((PIX_PER_TILE,), jnp.float32),
        ],
        compiler_params=pltpu.CompilerParams(needs_layout_passes=False),
    )(score2_flat, homo_pad)


def _shift(a, axis, s, fill):
    if s == 0:
        return a
    f = jnp.full_like(a, fill)
    n = a.shape[axis]
    if axis == 1:
        if s > 0:
            return jnp.concatenate([a[:, s:, :], f[:, :s, :]], axis=1)
        return jnp.concatenate([f[:, s:, :], a[:, :n + s, :]], axis=1)
    else:
        if s > 0:
            return jnp.concatenate([a[:, :, s:], f[:, :, :s]], axis=2)
        return jnp.concatenate([f[:, :, s:], a[:, :, :n + s]], axis=2)


def _win5_max(a, axis):
    neg = jnp.float32(-jnp.inf)
    out = a
    for s in (-2, -1, 1, 2):
        out = jnp.maximum(out, _shift(a, axis, s, neg))
    return out


def _gauss_taps():
    ax = np.arange(G_K, dtype=np.float32) - (G_K - 1) / 2.0
    g = np.exp(-(ax ** 2) / (2.0 * np.float32(G_SIGMA) ** 2)).astype(np.float32)
    g = g / g.sum()
    return [float(v) for v in g]


def _blur1(a, axis, taps):
    out = a * taps[2]
    for k, s in ((0, -2), (1, -1), (3, 1), (4, 2)):
        out = out + _shift(a, axis, s, 0.0) * taps[k]
    return out


def _tc_body(s1_ref, w2_ref, m_ref, out_ref):
    taps = _gauss_taps()
    w2 = w2_ref[...]
    pooled = _win5_max(_win5_max(w2, 2), 1)
    peak = (w2 == pooled) & (w2 > 0.0)
    masked = jnp.where(peak, w2, jnp.float32(0.0))
    bits = lax.bitcast_convert_type(masked, jnp.int32)

    def bisect(i, lohi):
        lo, hi = lohi
        mid = (lo + hi) // 2
        cnt = jnp.sum((bits >= mid).astype(jnp.int32), axis=(1, 2),
                      keepdims=True)
        big = cnt >= TOP_K
        return (jnp.where(big, mid, lo), jnp.where(big, hi, mid))

    lo0 = jnp.zeros((B, 1, 1), jnp.int32)
    hi0 = jnp.full((B, 1, 1), 0x3F800001, jnp.int32)
    lo, hi = lax.fori_loop(0, 26, bisect, (lo0, hi0))
    gt = ((bits >= lo) & (masked > 0.0)).astype(jnp.float32)
    g = _blur1(_blur1(gt, 2, taps), 1, taps)
    d = s1_ref[...] - g
    m = m_ref[...]
    num = jnp.sum(d * d * m)
    den = jnp.sum(m)
    out_ref[0, 0] = num * LAM / den


@jax.jit
def _tc_rest(s1, w2, mf):
    return pl.pallas_call(
        _tc_body,
        out_shape=jax.ShapeDtypeStruct((1, 1), jnp.float32),
        out_specs=pl.BlockSpec(memory_space=pltpu.SMEM),
    )(s1, w2, mf)


def kernel(score1, score2, w_vis_mask2, homo12):
    s2f = score2.reshape(B, HW)
    homo_pad = jnp.concatenate(
        [homo12.reshape(B, 9), jnp.zeros((B, 7), jnp.float32)], axis=1)
    w_flat = _sc_warp(s2f, homo_pad)
    w2 = w_flat.reshape(B, H, W)
    s1 = score1.reshape(B, H, W)
    mf = w_vis_mask2.astype(jnp.float32).reshape(B, H, W)
    loss = _tc_rest(s1, w2, mf)
    return loss[0, 0]


# speedup vs baseline: 42.5258x; 1.0162x over previous
import functools

import numpy as np
import jax
import jax.numpy as jnp
from jax import lax
from jax.experimental import pallas as pl
from jax.experimental.pallas import tpu as pltpu
from jax.experimental.pallas import tpu_sc as plsc

B, H, W = 8, 240, 320
HW = H * W
NMS_K = 5
TOP_K = 512
G_K = 5
G_SIGMA = 0.5
LAM = 1.0

NC, NS = 2, 16
NW = NC * NS
TILES_PER_IMG = NW // B
ROWS_PER_TILE = H // TILES_PER_IMG
PIX_PER_TILE = ROWS_PER_TILE * W
VECS_PER_ROW = W // 16
VECS_PER_TILE = PIX_PER_TILE // 16


def _warp_body(score2_hbm, homo_hbm, out_hbm, img_v, homo_v, out_v):
    cid = lax.axis_index("c")
    sid = lax.axis_index("s")
    wid = sid * NC + cid
    b = wid // TILES_PER_IMG
    quarter = wid % TILES_PER_IMG

    pltpu.sync_copy(score2_hbm.at[b], img_v)
    pltpu.sync_copy(homo_hbm.at[b], homo_v)

    lane = lax.iota(jnp.int32, 16)

    hv = homo_v[...]
    h00, h01, h02 = hv[0], hv[1], hv[2]
    h10, h11, h12 = hv[3], hv[4], hv[5]
    h20, h21, h22 = hv[6], hv[7], hv[8]

    r0 = quarter * ROWS_PER_TILE
    lanef = lane.astype(jnp.float32)

    @plsc.parallel_loop(0, VECS_PER_TILE, unroll=4)
    def body(i):
        y = r0 + i // VECS_PER_ROW
        xb = (i % VECS_PER_ROW) * 16
        yf = y.astype(jnp.float32)
        xf = xb.astype(jnp.float32) + lanef
        z = h20 * xf + (h21 * yf + h22)
        z = jnp.where(jnp.abs(z) < 1e-8, jnp.float32(1e-8), z)
        x2 = (h00 * xf + (h01 * yf + h02)) / z
        y2 = (h10 * xf + (h11 * yf + h12)) / z
        valid = (x2 >= 0.0) & (x2 < W - 1.0) & (y2 >= 0.0) & (y2 < H - 1.0)
        xs = jnp.clip(x2, 0.0, jnp.float32(W - 1))
        ys = jnp.clip(y2, 0.0, jnp.float32(H - 1))
        x0 = xs.astype(jnp.int32)
        y0 = ys.astype(jnp.int32)
        wx = xs - x0.astype(jnp.float32)
        wy = ys - y0.astype(jnp.float32)
        x1 = jnp.minimum(x0 + 1, W - 1)
        y1 = jnp.minimum(y0 + 1, H - 1)
        ra = y0 * W
        rb = y1 * W
        va = plsc.load_gather(img_v, [ra + x0])
        vb = plsc.load_gather(img_v, [ra + x1])
        vc = plsc.load_gather(img_v, [rb + x0])
        vd = plsc.load_gather(img_v, [rb + x1])
        owx = 1.0 - wx
        owy = 1.0 - wy
        val = (va * owx + vb * wx) * owy + (vc * owx + vd * wx) * wy
        val = jnp.where(valid, val, 0.0)
        out_v[pl.ds(i * 16, 16)] = val

    pltpu.sync_copy(out_v, out_hbm.at[pl.ds(wid * PIX_PER_TILE, PIX_PER_TILE)])


@jax.jit
def _sc_warp(score2_flat, homo_pad):
    mesh = plsc.VectorSubcoreMesh(core_axis_name="c", subcore_axis_name="s",
                                  num_cores=NC, num_subcores=NS)
    return pl.kernel(
        _warp_body,
        out_type=jax.ShapeDtypeStruct((B * HW,), jnp.float32),
        mesh=mesh,
        scratch_types=[
            pltpu.VMEM((HW,), jnp.float32),
            pltpu.VMEM((16,), jnp.float32),
            pltpu.VMEM((PIX_PER_TILE,), jnp.float32),
        ],
        compiler_params=pltpu.CompilerParams(needs_layout_passes=False),
    )(score2_flat, homo_pad)


def _shift(a, axis, s, fill):
    if s == 0:
        return a
    f = jnp.full_like(a, fill)
    n = a.shape[axis]
    if axis == 1:
        if s > 0:
            return jnp.concatenate([a[:, s:, :], f[:, :s, :]], axis=1)
        return jnp.concatenate([f[:, s:, :], a[:, :n + s, :]], axis=1)
    else:
        if s > 0:
            return jnp.concatenate([a[:, :, s:], f[:, :, :s]], axis=2)
        return jnp.concatenate([f[:, :, s:], a[:, :, :n + s]], axis=2)


def _win5_max(a, axis):
    neg = jnp.float32(-jnp.inf)
    out = a
    for s in (-2, -1, 1, 2):
        out = jnp.maximum(out, _shift(a, axis, s, neg))
    return out


def _gauss_taps():
    ax = np.arange(G_K, dtype=np.float32) - (G_K - 1) / 2.0
    g = np.exp(-(ax ** 2) / (2.0 * np.float32(G_SIGMA) ** 2)).astype(np.float32)
    g = g / g.sum()
    return [float(v) for v in g]


def _blur1(a, axis, taps):
    out = a * taps[2]
    for k, s in ((0, -2), (1, -1), (3, 1), (4, 2)):
        out = out + _shift(a, axis, s, 0.0) * taps[k]
    return out


def _tc_body(s1_ref, w2_ref, m_ref, out_ref):
    taps = _gauss_taps()
    w2 = w2_ref[...]
    pooled = _win5_max(_win5_max(w2, 2), 1)
    peak = (w2 == pooled) & (w2 > 0.0)
    masked = jnp.where(peak, w2, jnp.float32(0.0))
    bits = lax.bitcast_convert_type(masked, jnp.int32)

    def bisect(i, lohi):
        lo, hi = lohi
        mid = (lo + hi) // 2
        cnt = jnp.sum((bits >= mid).astype(jnp.int32), axis=(1, 2),
                      keepdims=True)
        big = cnt >= TOP_K
        return (jnp.where(big, mid, lo), jnp.where(big, hi, mid))

    lo0 = jnp.zeros((B, 1, 1), jnp.int32)
    hi0 = jnp.full((B, 1, 1), 0x3F800001, jnp.int32)
    lo, hi = lax.fori_loop(0, 26, bisect, (lo0, hi0))
    gt = ((bits >= lo) & (masked > 0.0)).astype(jnp.float32)
    g = _blur1(_blur1(gt, 2, taps), 1, taps)
    d = s1_ref[...] - g
    m = m_ref[...]
    num = jnp.sum(d * d * m)
    den = jnp.sum(m)
    out_ref[0, 0] = num * LAM / den


@jax.jit
def _tc_rest(s1, w2, mf):
    return pl.pallas_call(
        _tc_body,
        out_shape=jax.ShapeDtypeStruct((1, 1), jnp.float32),
        out_specs=pl.BlockSpec(memory_space=pltpu.SMEM),
    )(s1, w2, mf)


def kernel(score1, score2, w_vis_mask2, homo12):
    s2f = score2.reshape(B, HW)
    homo_pad = jnp.concatenate(
        [homo12.reshape(B, 9), jnp.zeros((B, 7), jnp.float32)], axis=1)
    w_flat = _sc_warp(s2f, homo_pad)
    w2 = w_flat.reshape(B, H, W)
    s1 = score1.reshape(B, H, W)
    mf = w_vis_mask2.astype(jnp.float32).reshape(B, H, W)
    loss = _tc_rest(s1, w2, mf)
    return loss[0, 0]
